# Initial kernel scaffold; baseline (speedup 1.0000x reference)
import jax
import jax.numpy as jnp
from jax.experimental import pallas as pl


def kernel(x, W_bin, W_pos):
    raise NotImplementedError("write your pallas kernel here")


# speedup vs baseline: 1.2991x; 1.2991x over previous
import functools

import jax
import jax.numpy as jnp
from jax import lax
from jax.experimental import pallas as pl
from jax.experimental.pallas import tpu as pltpu
from jax.experimental.pallas import tpu_sc as plsc

_NC = 2
_NS = 16
_NW = _NC * _NS
_L = 16


def _build_table(w_bin, w_pos):
    e_types, d = w_bin.shape
    p_rows = w_pos.shape[0]

    def body(wb_ref, wp_ref, t_ref):
        wp = wp_ref[...]
        for e in range(e_types):
            t_ref[pl.ds(e * p_rows, p_rows), :] = wp + wb_ref[e, :][None, :]

    return pl.pallas_call(
        body,
        out_shape=jax.ShapeDtypeStruct((e_types * p_rows, d), jnp.float32),
    )(w_bin, w_pos)


def _sc_expand(x2, table, n, d, p_rows, bin_size):
    rows = x2.shape[0]
    rpw = rows // _NW
    n_chunks = n // 128
    mesh = plsc.VectorSubcoreMesh(
        core_axis_name="c", subcore_axis_name="s",
        num_cores=_NC, num_subcores=_NS)

    @functools.partial(
        pl.kernel,
        out_type=jax.ShapeDtypeStruct((rows, n, d), jnp.float32),
        mesh=mesh,
        compiler_params=pltpu.CompilerParams(use_tc_tiling_on_sc=False),
        scratch_types=[
            pltpu.VMEM((n,), jnp.int32),
            pltpu.VMEM((n_chunks, 128), jnp.int32),
            pltpu.VMEM((n, d), jnp.float32),
            pltpu.SemaphoreType.DMA,
        ],
    )
    def run(x_hbm, t_hbm, out_hbm, xrow, idx, orow, sem):
        wid = lax.axis_index("s") * _NC + lax.axis_index("c")
        base = wid * rpw

        def row_body(r, carry):
            row = base + r
            i = lax.rem(row, n)
            pltpu.sync_copy(x_hbm.at[row], xrow)

            for kk in range(n_chunks):
                def idx_body(t, carry2, kk=kk):
                    jj = kk * 8 + t
                    xv = xrow[pl.ds(jj * _L, _L)]
                    jvec = jj * _L + lax.iota(jnp.int32, _L)
                    rel = jvec - i
                    p = jnp.maximum(jnp.minimum(rel, bin_size), -bin_size)
                    idx[kk, pl.ds(t * _L, _L)] = xv * p_rows + (p + bin_size)
                    return carry2
                lax.fori_loop(0, 128 // _L, idx_body, 0)

            copies = [
                pltpu.async_copy(
                    t_hbm.at[idx.at[kk]],
                    orow.at[pl.ds(kk * 128, 128)],
                    sem,
                )
                for kk in range(n_chunks)
            ]
            for cp in copies:
                cp.wait()
            pltpu.sync_copy(orow, out_hbm.at[row])
            return carry

        lax.fori_loop(0, rpw, row_body, 0)

    return run(x2, table)


def kernel(x, W_bin, W_pos):
    b, n = x.shape[0], x.shape[1]
    e_types, d = W_bin.shape
    p_rows = W_pos.shape[0]
    bin_size = (p_rows - 1) // 2

    x2 = x.reshape(b * n, n).astype(jnp.int32)
    table = _build_table(W_bin.astype(jnp.float32), W_pos.astype(jnp.float32))
    out = _sc_expand(x2, table, n, d, p_rows, bin_size)
    return out.reshape(b, n, n, d)


# speedup vs baseline: 5.9658x; 4.5924x over previous
import functools

import jax
import jax.numpy as jnp
from jax import lax
from jax.experimental import pallas as pl
from jax.experimental.pallas import tpu as pltpu
from jax.experimental.pallas import tpu_sc as plsc

_NC = 2
_NS = 16
_NW = _NC * _NS
_L = 16


def _sc_expand(x2, w_bin, w_pos, n, d, p_rows, bin_size, e_types):
    rows = x2.shape[0]
    rpw = rows // _NW
    nj = n // _L
    mesh = plsc.VectorSubcoreMesh(
        core_axis_name="c", subcore_axis_name="s",
        num_cores=_NC, num_subcores=_NS)

    @functools.partial(
        pl.kernel,
        out_type=jax.ShapeDtypeStruct((rows, n, d), jnp.float32),
        mesh=mesh,
        compiler_params=pltpu.CompilerParams(
            use_tc_tiling_on_sc=False, needs_layout_passes=False),
        scratch_types=[
            pltpu.VMEM((e_types, d), jnp.float32),
            pltpu.VMEM((p_rows, d), jnp.float32),
            pltpu.VMEM((e_types * p_rows, d), jnp.float32),
            pltpu.VMEM((rpw, n), jnp.int32),
            pltpu.VMEM((n, d), jnp.float32),
            pltpu.VMEM((n, d), jnp.float32),
            pltpu.SemaphoreType.DMA,
            pltpu.SemaphoreType.DMA,
        ],
    )
    def run(x_hbm, wb_hbm, wp_hbm, out_hbm,
            wb_v, wp_v, t_v, x_v, stga, stgb, sema, semb):
        wid = lax.axis_index("s") * _NC + lax.axis_index("c")
        base = wid * rpw
        pltpu.sync_copy(wb_hbm, wb_v)
        pltpu.sync_copy(wp_hbm, wp_v)
        pltpu.sync_copy(x_hbm.at[pl.ds(base, rpw)], x_v)

        io_lo = lax.iota(jnp.int32, _L)
        io_hi = io_lo + _L
        lane_of = [jnp.full((_L,), u, jnp.int32) for u in range(_L)]

        def lane_take(vec, idx):
            return lax.gather(
                vec, idx[:, None],
                lax.GatherDimensionNumbers(
                    offset_dims=(), collapsed_slice_dims=(0,),
                    start_index_map=(0,)),
                slice_sizes=(1,),
                mode=lax.GatherScatterMode.PROMISE_IN_BOUNDS)

        for e in range(e_types):
            wb_lo = wb_v[e, pl.ds(0, _L)]
            wb_hi = wb_v[e, pl.ds(_L, _L)]

            def table_body(p, carry, e=e, wb_lo=wb_lo, wb_hi=wb_hi):
                t_v[e * p_rows + p, pl.ds(0, _L)] = wp_v[p, pl.ds(0, _L)] + wb_lo
                t_v[e * p_rows + p, pl.ds(_L, _L)] = wp_v[p, pl.ds(_L, _L)] + wb_hi
                return carry

            lax.fori_loop(0, p_rows, table_body, 0)

        def compute_row(r, stg):
            i = lax.rem(base + r, n)

            def chunk(cc, carry):
                xv = x_v[r, pl.ds(cc * _L, _L)]
                rel = (cc * _L + io_lo) - i
                p = jnp.minimum(jnp.maximum(rel, -bin_size), bin_size)
                cvec = xv * p_rows + (p + bin_size)
                for u in range(_L):
                    bc = lane_take(cvec, lane_of[u])
                    j = cc * _L + u
                    stg[j, pl.ds(0, _L)] = plsc.load_gather(t_v, [bc, io_lo])
                    stg[j, pl.ds(_L, _L)] = plsc.load_gather(t_v, [bc, io_hi])
                return carry

            lax.fori_loop(0, nj, chunk, 0)

        def pair(q, carry):
            ra = 2 * q
            rb = ra + 1

            @pl.when(q > 0)
            def _wait_a():
                pltpu.make_async_copy(stga, out_hbm.at[base], sema).wait()

            compute_row(ra, stga)
            pltpu.async_copy(stga, out_hbm.at[base + ra], sema)

            @pl.when(q > 0)
            def _wait_b():
                pltpu.make_async_copy(stgb, out_hbm.at[base], semb).wait()

            compute_row(rb, stgb)
            pltpu.async_copy(stgb, out_hbm.at[base + rb], semb)
            return carry

        lax.fori_loop(0, rpw // 2, pair, 0)
        pltpu.make_async_copy(stga, out_hbm.at[base], sema).wait()
        pltpu.make_async_copy(stgb, out_hbm.at[base], semb).wait()

    return run(x2, w_bin, w_pos)


def kernel(x, W_bin, W_pos):
    b, n = x.shape[0], x.shape[1]
    e_types, d = W_bin.shape
    p_rows = W_pos.shape[0]
    bin_size = (p_rows - 1) // 2

    x2 = x.reshape(b * n, n).astype(jnp.int32)
    out = _sc_expand(x2, W_bin.astype(jnp.float32), W_pos.astype(jnp.float32),
                     n, d, p_rows, bin_size, e_types)
    return out.reshape(b, n, n, d)


# speedup vs baseline: 7.8319x; 1.3128x over previous
import functools

import jax
import jax.numpy as jnp
from jax import lax
from jax.experimental import pallas as pl
from jax.experimental.pallas import tpu as pltpu
from jax.experimental.pallas import tpu_sc as plsc

_NC = 2
_NS = 16
_NW = _NC * _NS
_L = 16


def _sc_expand(x2, w_bin, w_pos, n, d, p_rows, bin_size, e_types):
    rows = x2.shape[0]
    rpw = rows // _NW
    nj = n // _L
    mesh = plsc.VectorSubcoreMesh(
        core_axis_name="c", subcore_axis_name="s",
        num_cores=_NC, num_subcores=_NS)

    @functools.partial(
        pl.kernel,
        out_type=jax.ShapeDtypeStruct((rows, n, d), jnp.float32),
        mesh=mesh,
        compiler_params=pltpu.CompilerParams(
            use_tc_tiling_on_sc=False, needs_layout_passes=False),
        scratch_types=[
            pltpu.VMEM((e_types, d), jnp.float32),
            pltpu.VMEM((p_rows, d), jnp.float32),
            pltpu.VMEM((e_types * p_rows, d), jnp.float32),
            pltpu.VMEM_SHARED((e_types * p_rows, d), jnp.float32),
            pltpu.VMEM((rpw, n), jnp.int32),
            pltpu.VMEM((n, d), jnp.float32),
            pltpu.VMEM((n, d), jnp.float32),
            pltpu.VMEM((n // 128, 128), jnp.int32),
            pltpu.VMEM((n // 128, 128), jnp.int32),
            pltpu.SemaphoreType.DMA,
            pltpu.SemaphoreType.DMA,
            pltpu.SemaphoreType.DMA,
            pltpu.SemaphoreType.DMA,
        ],
    )
    def run(x_hbm, wb_hbm, wp_hbm, out_hbm,
            wb_v, wp_v, t_v, t_s, x_v, stga, stgb, idxa, idxb,
            sema, semb, gsema, gsemb):
        wid = lax.axis_index("s") * _NC + lax.axis_index("c")
        base = wid * rpw
        pltpu.sync_copy(wb_hbm, wb_v)
        pltpu.sync_copy(wp_hbm, wp_v)
        pltpu.sync_copy(x_hbm.at[pl.ds(base, rpw)], x_v)

        io_lo = lax.iota(jnp.int32, _L)

        for e in range(e_types):
            wb_lo = wb_v[e, pl.ds(0, _L)]
            wb_hi = wb_v[e, pl.ds(_L, _L)]

            def table_body(p, carry, e=e, wb_lo=wb_lo, wb_hi=wb_hi):
                t_v[e * p_rows + p, pl.ds(0, _L)] = wp_v[p, pl.ds(0, _L)] + wb_lo
                t_v[e * p_rows + p, pl.ds(_L, _L)] = wp_v[p, pl.ds(_L, _L)] + wb_hi
                return carry

            lax.fori_loop(0, p_rows, table_body, 0)

        @pl.when(lax.axis_index("s") == 0)
        def _publish():
            pltpu.sync_copy(t_v, t_s)

        plsc.subcore_barrier()

        nk = n // 128

        def compute_idx(r, idxv):
            i = lax.rem(base + r, n)
            for kk in range(nk):
                def chunk(t, carry, kk=kk):
                    cc = kk * (128 // _L) + t
                    xv = x_v[r, pl.ds(cc * _L, _L)]
                    rel = (cc * _L + io_lo) - i
                    p = jnp.minimum(jnp.maximum(rel, -bin_size), bin_size)
                    idxv[kk, pl.ds(t * _L, _L)] = xv * p_rows + (p + bin_size)
                    return carry
                lax.fori_loop(0, 128 // _L, chunk, 0)

        def fire_gathers(idxv, stg, gsem):
            for kk in range(nk):
                pltpu.async_copy(
                    t_s.at[idxv.at[kk]], stg.at[pl.ds(kk * 128, 128)], gsem)

        def drain_gathers(stg, gsem):
            pltpu.make_async_copy(out_hbm.at[base], stg, gsem).wait()

        def pair(q, carry):
            ra = 2 * q
            rb = ra + 1

            @pl.when(q > 0)
            def _wait_a():
                pltpu.make_async_copy(stga, out_hbm.at[base], sema).wait()

            compute_idx(ra, idxa)
            fire_gathers(idxa, stga, gsema)

            @pl.when(q > 0)
            def _wait_b():
                pltpu.make_async_copy(stgb, out_hbm.at[base], semb).wait()

            compute_idx(rb, idxb)
            drain_gathers(stga, gsema)
            pltpu.async_copy(stga, out_hbm.at[base + ra], sema)
            fire_gathers(idxb, stgb, gsemb)
            drain_gathers(stgb, gsemb)
            pltpu.async_copy(stgb, out_hbm.at[base + rb], semb)
            return carry

        lax.fori_loop(0, rpw // 2, pair, 0)
        pltpu.make_async_copy(stga, out_hbm.at[base], sema).wait()
        pltpu.make_async_copy(stgb, out_hbm.at[base], semb).wait()

    return run(x2, w_bin, w_pos)


def kernel(x, W_bin, W_pos):
    b, n = x.shape[0], x.shape[1]
    e_types, d = W_bin.shape
    p_rows = W_pos.shape[0]
    bin_size = (p_rows - 1) // 2

    x2 = x.reshape(b * n, n).astype(jnp.int32)
    out = _sc_expand(x2, W_bin.astype(jnp.float32), W_pos.astype(jnp.float32),
                     n, d, p_rows, bin_size, e_types)
    return out.reshape(b, n, n, d)
